# baseline (speedup 1.0000x reference)
import jax
import jax.numpy as jnp
from jax import lax
from jax.experimental import pallas as pl
from jax.experimental.pallas import tpu as pltpu

_B = 16384
_N = 6
_M = 20
_BN = 4096


def _tc_body(w_ref, ft_ref, out_ref):
    out_ref[...] = lax.dot_general(
        w_ref[...], ft_ref[...], (((0,), (0,)), ((), ())),
        preferred_element_type=jnp.float32,
    )


def kernel(f_values, W):
    out_t = pl.pallas_call(
        _tc_body,
        grid=(_B // _BN,),
        in_specs=[
            pl.BlockSpec((_N, _M), lambda i: (0, 0)),
            pl.BlockSpec((_N, _BN), lambda i: (0, i)),
        ],
        out_specs=pl.BlockSpec((_M, _BN), lambda i: (0, i)),
        out_shape=jax.ShapeDtypeStruct((_M, _B), jnp.float32),
    )(W, f_values.T)
    return out_t.T


# speedup vs baseline: 1.5439x; 1.5439x over previous
import jax
import jax.numpy as jnp
from jax import lax
from jax.experimental import pallas as pl
from jax.experimental.pallas import tpu as pltpu

_B = 16384
_N = 6
_M = 20
_BN = 8192


def _tc_body(w_ref, ft_ref, out_ref):
    out_ref[...] = lax.dot_general(
        w_ref[...], ft_ref[...], (((0,), (0,)), ((), ())),
        preferred_element_type=jnp.float32,
    )


def kernel(f_values, W):
    out_t = pl.pallas_call(
        _tc_body,
        grid=(_B // _BN,),
        in_specs=[
            pl.BlockSpec((_N, _M), lambda i: (0, 0)),
            pl.BlockSpec((_N, _BN), lambda i: (0, i)),
        ],
        out_specs=pl.BlockSpec((_M, _BN), lambda i: (0, i)),
        out_shape=jax.ShapeDtypeStruct((_M, _B), jnp.float32),
    )(W, f_values.T)
    return out_t.T


# speedup vs baseline: 1.6144x; 1.0457x over previous
import jax
import jax.numpy as jnp
from jax import lax
from jax.experimental import pallas as pl
from jax.experimental.pallas import tpu as pltpu

_B = 16384
_N = 6
_M = 20
_BN = 8192
_C = _B // _BN


def _tc_body(w_hbm, ft_hbm, out_hbm, w_v, ft_v, out_v, w_sem, in_sems,
             out_sems):
    w_cp = pltpu.make_async_copy(w_hbm, w_v, w_sem)
    w_cp.start()
    in_cps = [
        pltpu.make_async_copy(ft_hbm.at[:, pl.ds(i * _BN, _BN)], ft_v.at[i],
                              in_sems.at[i])
        for i in range(_C)
    ]
    out_cps = [
        pltpu.make_async_copy(out_v.at[i], out_hbm.at[:, pl.ds(i * _BN, _BN)],
                              out_sems.at[i])
        for i in range(_C)
    ]
    for cp in in_cps:
        cp.start()
    w_cp.wait()
    w = w_v[...]
    for i in range(_C):
        in_cps[i].wait()
        out_v[i] = lax.dot_general(
            w, ft_v[i], (((0,), (0,)), ((), ())),
            preferred_element_type=jnp.float32,
        )
        out_cps[i].start()
    for cp in out_cps:
        cp.wait()


def kernel(f_values, W):
    out_t = pl.pallas_call(
        _tc_body,
        in_specs=[
            pl.BlockSpec(memory_space=pltpu.MemorySpace.HBM),
            pl.BlockSpec(memory_space=pltpu.MemorySpace.HBM),
        ],
        out_specs=pl.BlockSpec(memory_space=pltpu.MemorySpace.HBM),
        out_shape=jax.ShapeDtypeStruct((_M, _B), jnp.float32),
        scratch_shapes=[
            pltpu.VMEM((_N, _M), jnp.float32),
            pltpu.VMEM((_C, _N, _BN), jnp.float32),
            pltpu.VMEM((_C, _M, _BN), jnp.float32),
            pltpu.SemaphoreType.DMA,
            pltpu.SemaphoreType.DMA((_C,)),
            pltpu.SemaphoreType.DMA((_C,)),
        ],
    )(W, f_values.T)
    return out_t.T
